# baseline (speedup 1.0000x reference)
import functools

import jax
import jax.numpy as jnp
from jax import lax
from jax.experimental import pallas as pl
from jax.experimental.pallas import tpu as pltpu
from jax.experimental.pallas import tpu_sc as plsc

N = 10000
E = 320000
H = 128

NC = 2
NS = 16
NW = NC * NS
L = 16

CH = 80
PER_W = E // NW
NCHUNK = PER_W // CH
GCH = 128
GFULL = PER_W // GCH
GTAIL = PER_W - GFULL * GCH
ROWS_PER_TILE = N // 10

_EPS = 1e-5


def _ln(h, g, b):
    m = jnp.mean(h, axis=-1, keepdims=True)
    v = jnp.mean((h - m) * (h - m), axis=-1, keepdims=True)
    return (h - m) * lax.rsqrt(v + _EPS) * g + b


def _bdot(a, b):
    return jnp.dot(a.astype(jnp.bfloat16), b.astype(jnp.bfloat16),
                   preferred_element_type=jnp.float32)


def _pre_body(x_ref, w1a_ref, w1b_ref, b1_ref, xs_ref, xr_ref):
    xb = x_ref[...]
    xs_ref[...] = _bdot(xb, w1a_ref[...])
    xr_ref[...] = _bdot(xb, w1b_ref[...]) + b1_ref[...]


def _edge_body(g_ref, ea_ref, w1c_ref, w2_ref, b2_ref, gam_ref, bet_ref,
               enew_ref, eout_ref):
    ea = ea_ref[...]
    h = jnp.maximum(g_ref[...] + _bdot(ea, w1c_ref[...]), 0.0)
    h2 = _bdot(h, w2_ref[...]) + b2_ref[...]
    en = _ln(h2, gam_ref[...], bet_ref[...])
    enew_ref[...] = en
    eout_ref[...] = ea + en


def _node_body(x_ref, p0_ref, p1_ref, w1x_ref, w1a_ref, b1_ref, w2_ref, b2_ref,
               gam_ref, bet_ref, out_ref):
    xb = x_ref[...]
    agg = p0_ref[...] + p1_ref[...]
    h = jnp.maximum(
        _bdot(xb, w1x_ref[...]) + _bdot(agg, w1a_ref[...]) + b1_ref[...], 0.0)
    h2 = _bdot(h, w2_ref[...]) + b2_ref[...]
    out_ref[...] = xb + _ln(h2, gam_ref[...], bet_ref[...])


def _full(shape):
    return pl.BlockSpec(shape, lambda i: (0,) * len(shape))


_MESH = plsc.VectorSubcoreMesh(core_axis_name="c", subcore_axis_name="s")


@functools.partial(
    pl.kernel,
    out_type=jax.ShapeDtypeStruct((E, H), jnp.float32),
    mesh=_MESH,
    scratch_types=[
        pltpu.VMEM((1, PER_W), jnp.int32),
        pltpu.VMEM((1, PER_W), jnp.int32),
        pltpu.VMEM((2, GCH, H), jnp.float32),
        pltpu.VMEM((2, GCH, H), jnp.float32),
        pltpu.SemaphoreType.DMA,
        pltpu.SemaphoreType.DMA,
    ],
)
def _gather_combine(xs_hbm, xr_hbm, sidx_hbm, ridx_hbm, g_hbm,
                    sidx_v, ridx_v, buf_a, buf_b, gsem, wsem):
    wid = lax.axis_index("s") * NC + lax.axis_index("c")
    base0 = wid * PER_W
    pltpu.sync_copy(sidx_hbm.at[wid], sidx_v)
    pltpu.sync_copy(ridx_hbm.at[wid], ridx_v)

    def fire(j, slot):
        sl = pl.ds(j * GCH, GCH)
        pltpu.async_copy(xs_hbm.at[sidx_v.at[0, sl]], buf_a.at[slot], gsem)
        pltpu.async_copy(xr_hbm.at[ridx_v.at[0, sl]], buf_b.at[slot], gsem)

    def wait_gather(slot):
        pltpu.make_async_copy(xs_hbm.at[pl.ds(0, GCH)], buf_a.at[slot], gsem).wait()
        pltpu.make_async_copy(xs_hbm.at[pl.ds(0, GCH)], buf_b.at[slot], gsem).wait()

    def wait_write(slot):
        pltpu.make_async_copy(buf_a.at[slot], g_hbm.at[pl.ds(0, GCH)], wsem).wait()

    def add_rows(slot, nrows):
        def add_row(i, c):
            for k in range(H // L):
                s = pl.ds(k * L, L)
                buf_a[slot, i, s] = buf_a[slot, i, s] + buf_b[slot, i, s]
            return c
        lax.fori_loop(0, nrows, add_row, 0)

    fire(0, 0)

    def chunk(j, carry):
        r = lax.rem(j, 2)
        nxt = 1 - r

        wait_gather(r)

        @pl.when(j >= 1)
        def _():
            wait_write(nxt)

        @pl.when(j < GFULL - 1)
        def _():
            fire(j + 1, nxt)

        add_rows(r, GCH)
        pltpu.async_copy(buf_a.at[r], g_hbm.at[pl.ds(base0 + j * GCH, GCH)], wsem)
        return carry

    lax.fori_loop(0, GFULL, chunk, 0)
    wait_write((GFULL - 1) % 2)

    tsl = pl.ds(GFULL * GCH, GTAIL)
    ca = pltpu.async_copy(xs_hbm.at[sidx_v.at[0, tsl]], buf_a.at[0, pl.ds(0, GTAIL)], gsem)
    cb = pltpu.async_copy(xr_hbm.at[ridx_v.at[0, tsl]], buf_b.at[0, pl.ds(0, GTAIL)], gsem)
    ca.wait()
    cb.wait()
    add_rows(0, GTAIL)
    pltpu.sync_copy(buf_a.at[0, pl.ds(0, GTAIL)],
                    g_hbm.at[pl.ds(base0 + GFULL * GCH, GTAIL)])


@functools.partial(
    pl.kernel,
    out_type=jax.ShapeDtypeStruct((NC, N, H), jnp.float32),
    mesh=_MESH,
    scratch_types=[
        pltpu.VMEM((NCHUNK, CH), jnp.int32),
        pltpu.VMEM((2, CH, H), jnp.float32),
        pltpu.VMEM((ROWS_PER_TILE // 25, H), jnp.float32),
        pltpu.VMEM_SHARED((N, H), jnp.float32),
        pltpu.SemaphoreType.DMA,
    ],
)
def _scatter_add(enew_hbm, ridx_hbm, out_hbm, ridx_v, rows_v, zbuf, acc, sem):
    cid = lax.axis_index("c")
    sid = lax.axis_index("s")
    wid = sid * NC + cid
    row0 = wid * NCHUNK

    zrows = ROWS_PER_TILE // 25

    def zero_row(i, c):
        for k in range(H // L):
            zbuf[i, pl.ds(k * L, L)] = jnp.zeros((L,), jnp.float32)
        return c

    lax.fori_loop(0, zrows, zero_row, 0)

    @pl.when(sid < 10)
    def _():
        for t in range(25):
            pltpu.sync_copy(
                zbuf, acc.at[pl.ds(sid * ROWS_PER_TILE + t * zrows, zrows)])

    plsc.subcore_barrier()

    pltpu.sync_copy(ridx_hbm.at[wid], ridx_v)

    def fetch(j, slot):
        pltpu.async_copy(enew_hbm.at[pl.ds((row0 + j) * CH, CH)],
                         rows_v.at[slot], sem)

    fetch(0, 0)

    def chunk(j, carry):
        r = lax.rem(j, 2)
        pltpu.make_async_copy(enew_hbm.at[pl.ds(0, CH)], rows_v.at[r], sem).wait()

        @pl.when(j < NCHUNK - 1)
        def _():
            fetch(j + 1, 1 - r)

        pltpu.sync_copy(rows_v.at[r], acc.at[ridx_v.at[j]], add=True)
        return carry

    lax.fori_loop(0, NCHUNK, chunk, 0)
    plsc.subcore_barrier()

    @pl.when(sid < 10)
    def _():
        pltpu.sync_copy(acc.at[pl.ds(sid * ROWS_PER_TILE, ROWS_PER_TILE)],
                        out_hbm.at[cid, pl.ds(sid * ROWS_PER_TILE, ROWS_PER_TILE)])


def kernel(x, edge_attr, edge_index, eb_W1, eb_b1, eb_W2, eb_b2, eb_g, eb_beta,
           nb_W1, nb_b1, nb_W2, nb_b2, nb_g, nb_beta):
    senders = edge_index[0].astype(jnp.int32).reshape(NW, 1, PER_W)
    receivers_g = edge_index[1].astype(jnp.int32).reshape(NW, 1, PER_W)
    receivers = edge_index[1].astype(jnp.int32).reshape(NW, NCHUNK, CH)

    w1a, w1b, w1c = eb_W1[:H], eb_W1[H:2 * H], eb_W1[2 * H:]
    nw1x, nw1a = nb_W1[:H], nb_W1[H:]

    nb_blk = 1000
    xs, xr = pl.pallas_call(
        _pre_body,
        grid=(N // nb_blk,),
        in_specs=[
            pl.BlockSpec((nb_blk, H), lambda i: (i, 0)),
            _full((H, H)), _full((H, H)), _full((H,)),
        ],
        out_specs=[pl.BlockSpec((nb_blk, H), lambda i: (i, 0))] * 2,
        out_shape=[jax.ShapeDtypeStruct((N, H), jnp.float32)] * 2,
    )(x, w1a, w1b, eb_b1)

    g = _gather_combine(xs, xr, senders, receivers_g)

    eb_blk = 2000
    e_new, e_out = pl.pallas_call(
        _edge_body,
        grid=(E // eb_blk,),
        in_specs=[
            pl.BlockSpec((eb_blk, H), lambda i: (i, 0)),
            pl.BlockSpec((eb_blk, H), lambda i: (i, 0)),
            _full((H, H)), _full((H, H)), _full((H,)), _full((H,)), _full((H,)),
        ],
        out_specs=[pl.BlockSpec((eb_blk, H), lambda i: (i, 0))] * 2,
        out_shape=[jax.ShapeDtypeStruct((E, H), jnp.float32)] * 2,
    )(g, edge_attr, w1c, eb_W2, eb_b2, eb_g, eb_beta)

    partial = _scatter_add(e_new, receivers)

    x_out = pl.pallas_call(
        _node_body,
        grid=(N // nb_blk,),
        in_specs=[
            pl.BlockSpec((nb_blk, H), lambda i: (i, 0)),
            pl.BlockSpec((nb_blk, H), lambda i: (i, 0)),
            pl.BlockSpec((nb_blk, H), lambda i: (i, 0)),
            _full((H, H)), _full((H, H)), _full((H,)),
            _full((H, H)), _full((H,)), _full((H,)), _full((H,)),
        ],
        out_specs=pl.BlockSpec((nb_blk, H), lambda i: (i, 0)),
        out_shape=jax.ShapeDtypeStruct((N, H), jnp.float32),
    )(x, partial[0], partial[1], nw1x, nw1a, nb_b1, nb_W2, nb_b2, nb_g, nb_beta)

    return (x_out, e_out)
